# baseline (speedup 1.0000x reference)
import functools

import jax
import jax.numpy as jnp
from jax import lax
from jax.experimental import pallas as pl
from jax.experimental.pallas import tpu as pltpu
from jax.experimental.pallas import tpu_sc as plsc

N = 10000
E = 320000
D = 128

NC = 2
NS = 16
NW = NC * NS

NPAD = 10240

K3_CHUNK = 64
K3_NCHUNK = 160
E3_PER_TILE = K3_CHUNK * K3_NCHUNK
NBUF = 4
IBLK = 32

ROWS_PER_TILE = NPAD // NS


def _zero_1d(ref, nwords):
    zeros = jnp.zeros((16,), ref.dtype)

    def body(i, _):
        ref[pl.ds(i * 16, 16)] = zeros
        return _

    lax.fori_loop(0, nwords // 16, body, None)


def _k1_body(dst_hbm, hist_hbm, dstv, histv, scrv):
    c = lax.axis_index("c")
    s = lax.axis_index("s")
    w = s * NC + c

    pltpu.sync_copy(dst_hbm.at[w], dstv)
    _zero_1d(histv, NPAD)

    iota = lax.iota(jnp.int32, 16)
    prev_idx = jnp.maximum(iota - 1, 0)
    next_idx = jnp.minimum(iota + 1, 15)

    def chunk_body(j, _):
        for k in range(K3_CHUNK // 16):
            d = dstv[j, pl.ds(k * 16, 16)]
            s_sorted, _v = plsc.sort_key_val(d, d)
            scrv[pl.ds(0, 16)] = s_sorted
            prev = plsc.load_gather(scrv, [prev_idx])
            nxt = plsc.load_gather(scrv, [next_idx])
            first = (s_sorted != prev) | (iota == 0)
            last = (s_sorted != nxt) | (iota == 15)
            fs = plsc.cummax(jnp.where(first, iota, jnp.zeros_like(iota)))
            cnt = (iota - fs + 1).astype(jnp.float32)
            plsc.addupdate_scatter(histv, [s_sorted], cnt, mask=last)
        return _

    lax.fori_loop(0, K3_NCHUNK, chunk_body, None)
    pltpu.sync_copy(histv, hist_hbm.at[w])


def _k3_body(src_hbm, dst_hbm, fpre_hbm, h0_hbm, h1_hbm,
             sidx, didx, b0, b1, b2, b3, zrow, acc_sh,
             g0, g1, g2, g3, s0, s1, s2, s3):
    bufs = [b0, b1, b2, b3]
    gsems = [g0, g1, g2, g3]
    ssems = [s0, s1, s2, s3]
    c = lax.axis_index("c")
    s = lax.axis_index("s")
    w = s * NC + c

    zeros = jnp.zeros((16,), jnp.float32)
    for i in range(16):
        for k in range(D // 16):
            zrow[i, pl.ds(k * 16, 16)] = zeros

    def zb(b, _):
        pltpu.sync_copy(zrow, acc_sh.at[pl.ds(s * ROWS_PER_TILE + b * 16, 16)])
        return _

    lax.fori_loop(0, ROWS_PER_TILE // 16, zb, None)
    plsc.subcore_barrier()

    def start_g(k, j):
        pltpu.async_copy(fpre_hbm.at[sidx.at[j]], bufs[k], gsems[k])

    def wait_g(k, j):
        pltpu.make_async_copy(fpre_hbm.at[sidx.at[j]], bufs[k],
                              gsems[k]).wait()

    def start_s(k, j):
        pltpu.make_async_copy(bufs[k], acc_sh.at[didx.at[j]],
                              ssems[k]).start(add=True)

    def wait_s(k, j):
        pltpu.make_async_copy(bufs[k], acc_sh.at[didx.at[j]],
                              ssems[k]).wait()

    NG = IBLK // NBUF
    for b in range(K3_NCHUNK // IBLK):
        pltpu.sync_copy(src_hbm.at[w, pl.ds(b * IBLK, IBLK)], sidx)
        pltpu.sync_copy(dst_hbm.at[w, pl.ds(b * IBLK, IBLK)], didx)
        for k in range(NBUF):
            start_g(k, k)

        wait_g(0, 0); start_s(0, 0)
        wait_g(1, 1); start_s(1, 1)
        wait_g(2, 2); start_s(2, 2)
        wait_g(3, 3); start_s(3, 3)
        wait_s(0, 0); start_g(0, 4)
        wait_s(1, 1); start_g(1, 5)

        def mbody(ii, _):
            i0 = NBUF * ii
            wait_g(0, i0); start_s(0, i0)
            wait_g(1, i0 + 1); start_s(1, i0 + 1)
            wait_s(2, i0 - 2); start_g(2, i0 + 2)
            wait_s(3, i0 - 1); start_g(3, i0 + 3)
            wait_g(2, i0 + 2); start_s(2, i0 + 2)
            wait_g(3, i0 + 3); start_s(3, i0 + 3)
            wait_s(0, i0); start_g(0, i0 + 4)
            wait_s(1, i0 + 1); start_g(1, i0 + 5)
            return _

        lax.fori_loop(1, NG - 1, mbody, None)

        i0 = IBLK - NBUF
        wait_g(0, i0); start_s(0, i0)
        wait_g(1, i0 + 1); start_s(1, i0 + 1)
        wait_s(2, i0 - 2); start_g(2, i0 + 2)
        wait_s(3, i0 - 1); start_g(3, i0 + 3)
        wait_g(2, i0 + 2); start_s(2, i0 + 2)
        wait_g(3, i0 + 3); start_s(3, i0 + 3)
        wait_s(0, i0)
        wait_s(1, i0 + 1)
        wait_s(2, i0 + 2)
        wait_s(3, i0 + 3)
    plsc.subcore_barrier()

    roff = s * ROWS_PER_TILE

    @pl.when(c == 0)
    def _():
        pltpu.sync_copy(acc_sh.at[pl.ds(roff, ROWS_PER_TILE)],
                        h0_hbm.at[pl.ds(roff, ROWS_PER_TILE)])

    @pl.when(c == 1)
    def _():
        pltpu.sync_copy(acc_sh.at[pl.ds(roff, ROWS_PER_TILE)],
                        h1_hbm.at[pl.ds(roff, ROWS_PER_TILE)])


_ROWBLK = 2048
_GRID = NPAD // _ROWBLK


def _norm_col(norm2d):
    nq = _ROWBLK // D
    e1 = (lax.broadcasted_iota(jnp.int32, (_ROWBLK, nq), 1)
          == lax.broadcasted_iota(jnp.int32, (_ROWBLK, nq), 0) // D
          ).astype(jnp.float32)
    t = jnp.dot(e1, norm2d, preferred_element_type=jnp.float32)
    sel = (lax.broadcasted_iota(jnp.int32, (_ROWBLK, D), 1)
           == lax.broadcasted_iota(jnp.int32, (_ROWBLK, D), 0) % D
           ).astype(jnp.float32)
    return jnp.sum(t * sel, axis=1, keepdims=True)


def _k2_body(hist_ref, feat_ref, fpre_ref, norm_ref):
    deg = jnp.sum(hist_ref[...], axis=0)
    norm2d = lax.rsqrt(jnp.maximum(deg, 1.0))
    norm_ref[...] = norm2d
    fpre_ref[...] = feat_ref[...] * _norm_col(norm2d)


def _k2(hist3, features):
    return pl.pallas_call(
        _k2_body,
        grid=(_GRID,),
        in_specs=[
            pl.BlockSpec((NW, _ROWBLK // D, D), lambda i: (0, i, 0)),
            pl.BlockSpec((_ROWBLK, D), lambda i: (i, 0)),
        ],
        out_specs=[
            pl.BlockSpec((_ROWBLK, D), lambda i: (i, 0)),
            pl.BlockSpec((_ROWBLK // D, D), lambda i: (i, 0)),
        ],
        out_shape=[
            jax.ShapeDtypeStruct((NPAD, D), jnp.float32),
            jax.ShapeDtypeStruct((NPAD // D, D), jnp.float32),
        ],
    )(hist3, features)


def _k4_body(h0_ref, h1_ref, norm_ref, feat_ref, init_ref, aw_ref, out_ref):
    h = (h0_ref[...] + h1_ref[...]) * _norm_col(norm_ref[...])
    f = feat_ref[...]
    a1 = aw_ref[0:1, 0:D]
    a2 = aw_ref[0:1, D:2 * D]
    logit = (jnp.sum(f * a1, axis=1, keepdims=True)
             + jnp.sum(h * a2, axis=1, keepdims=True))
    alpha = jax.nn.sigmoid(logit)
    out_ref[...] = alpha * h + init_ref[...]


def _k4(h0, h1, norm2d, features, initial_features, a_weight):
    blk = lambda i: (i, 0)
    return pl.pallas_call(
        _k4_body,
        grid=(_GRID,),
        in_specs=[
            pl.BlockSpec((_ROWBLK, D), blk),
            pl.BlockSpec((_ROWBLK, D), blk),
            pl.BlockSpec((_ROWBLK // D, D), blk),
            pl.BlockSpec((_ROWBLK, D), blk),
            pl.BlockSpec((_ROWBLK, D), blk),
            pl.BlockSpec((1, 2 * D), lambda i: (0, 0)),
        ],
        out_specs=pl.BlockSpec((_ROWBLK, D), blk),
        out_shape=jax.ShapeDtypeStruct((N, D), jnp.float32),
    )(h0, h1, norm2d, features, initial_features, a_weight)


_sc_mesh = plsc.VectorSubcoreMesh(core_axis_name="c", subcore_axis_name="s")

_k1 = functools.partial(
    pl.kernel,
    out_type=jax.ShapeDtypeStruct((NW, NPAD), jnp.float32),
    mesh=_sc_mesh,
    compiler_params=pltpu.CompilerParams(needs_layout_passes=False),
    scratch_types=[
        pltpu.VMEM((K3_NCHUNK, K3_CHUNK), jnp.int32),
        pltpu.VMEM((NPAD,), jnp.float32),
        pltpu.VMEM((16,), jnp.int32),
    ],
)(_k1_body)

_k3 = functools.partial(
    pl.kernel,
    out_type=(
        jax.ShapeDtypeStruct((NPAD, D), jnp.float32),
        jax.ShapeDtypeStruct((NPAD, D), jnp.float32),
    ),
    mesh=_sc_mesh,
    scratch_types=[
        pltpu.VMEM((IBLK, K3_CHUNK), jnp.int32),
        pltpu.VMEM((IBLK, K3_CHUNK), jnp.int32),
        pltpu.VMEM((K3_CHUNK, D), jnp.float32),
        pltpu.VMEM((K3_CHUNK, D), jnp.float32),
        pltpu.VMEM((K3_CHUNK, D), jnp.float32),
        pltpu.VMEM((K3_CHUNK, D), jnp.float32),
        pltpu.VMEM((16, D), jnp.float32),
        pltpu.VMEM_SHARED((NPAD, D), jnp.float32),
        pltpu.SemaphoreType.DMA,
        pltpu.SemaphoreType.DMA,
        pltpu.SemaphoreType.DMA,
        pltpu.SemaphoreType.DMA,
        pltpu.SemaphoreType.DMA,
        pltpu.SemaphoreType.DMA,
        pltpu.SemaphoreType.DMA,
        pltpu.SemaphoreType.DMA,
    ],
)(_k3_body)


def kernel(features, initial_features, edge_index, a_weight):
    src = edge_index[0]
    dst = edge_index[1]

    pad3 = NW * E3_PER_TILE - E
    dummy_dst = N + jnp.arange(pad3, dtype=jnp.int32) % (NPAD - N)
    dummy_src = jnp.arange(pad3, dtype=jnp.int32) % N
    src3 = jnp.concatenate(
        [src, dummy_src]).reshape(NW, K3_NCHUNK, K3_CHUNK)
    dst3 = jnp.concatenate(
        [dst, dummy_dst]).reshape(NW, K3_NCHUNK, K3_CHUNK)

    hist = _k1(dst3)
    fpre, norm2d = _k2(hist.reshape(NW, NPAD // D, D), features)
    h0, h1 = _k3(src3, dst3, fpre)
    out = _k4(h0, h1, norm2d, features, initial_features, a_weight)
    return out


# speedup vs baseline: 1.1087x; 1.1087x over previous
import functools

import jax
import jax.numpy as jnp
from jax import lax
from jax.experimental import pallas as pl
from jax.experimental.pallas import tpu as pltpu
from jax.experimental.pallas import tpu_sc as plsc

N = 10000
E = 320000
D = 128

NC = 2
NS = 16
NW = NC * NS

NPAD = 10240

K3_CHUNK = 128
K3_NCHUNK = 80
E3_PER_TILE = K3_CHUNK * K3_NCHUNK
IBLK = 16

ROWS_PER_TILE = NPAD // NS


def _zero_1d(ref, nwords):
    zeros = jnp.zeros((16,), ref.dtype)

    def body(i, _):
        ref[pl.ds(i * 16, 16)] = zeros
        return _

    lax.fori_loop(0, nwords // 16, body, None)


def _k1_body(dst_hbm, hist_hbm, dstv, histv, scrv):
    c = lax.axis_index("c")
    s = lax.axis_index("s")
    w = s * NC + c

    pltpu.sync_copy(dst_hbm.at[w], dstv)
    _zero_1d(histv, NPAD)

    iota = lax.iota(jnp.int32, 16)
    prev_idx = jnp.maximum(iota - 1, 0)
    next_idx = jnp.minimum(iota + 1, 15)

    def chunk_body(j, _):
        for k in range(K3_CHUNK // 16):
            d = dstv[j, pl.ds(k * 16, 16)]
            s_sorted, _v = plsc.sort_key_val(d, d)
            scrv[pl.ds(0, 16)] = s_sorted
            prev = plsc.load_gather(scrv, [prev_idx])
            nxt = plsc.load_gather(scrv, [next_idx])
            first = (s_sorted != prev) | (iota == 0)
            last = (s_sorted != nxt) | (iota == 15)
            fs = plsc.cummax(jnp.where(first, iota, jnp.zeros_like(iota)))
            cnt = (iota - fs + 1).astype(jnp.float32)
            plsc.addupdate_scatter(histv, [s_sorted], cnt, mask=last)
        return _

    lax.fori_loop(0, K3_NCHUNK, chunk_body, None)
    pltpu.sync_copy(histv, hist_hbm.at[w])


def _k3_body(src_hbm, dst_hbm, fpre_hbm, h0_hbm, h1_hbm,
             sidx, didx, rowsv, rows2v, zrow, acc_sh, gsem, gsem2):
    c = lax.axis_index("c")
    s = lax.axis_index("s")
    w = s * NC + c

    zeros = jnp.zeros((16,), jnp.float32)
    for i in range(16):
        for k in range(D // 16):
            zrow[i, pl.ds(k * 16, 16)] = zeros

    def zb(b, _):
        pltpu.sync_copy(zrow, acc_sh.at[pl.ds(s * ROWS_PER_TILE + b * 16, 16)])
        return _

    lax.fori_loop(0, ROWS_PER_TILE // 16, zb, None)
    plsc.subcore_barrier()

    for b in range(K3_NCHUNK // IBLK):
        pltpu.sync_copy(src_hbm.at[w, pl.ds(b * IBLK, IBLK)], sidx)
        pltpu.sync_copy(dst_hbm.at[w, pl.ds(b * IBLK, IBLK)], didx)
        pltpu.async_copy(fpre_hbm.at[sidx.at[0]], rowsv, gsem)
        pltpu.async_copy(fpre_hbm.at[sidx.at[1]], rows2v, gsem2)

        def mbody(ii, _):
            i0 = 2 * ii
            pltpu.make_async_copy(fpre_hbm.at[sidx.at[i0]], rowsv,
                                  gsem).wait()
            pltpu.sync_copy(rowsv, acc_sh.at[didx.at[i0]], add=True)
            pltpu.async_copy(fpre_hbm.at[sidx.at[i0 + 2]], rowsv, gsem)
            pltpu.make_async_copy(fpre_hbm.at[sidx.at[i0 + 1]], rows2v,
                                  gsem2).wait()
            pltpu.sync_copy(rows2v, acc_sh.at[didx.at[i0 + 1]], add=True)
            pltpu.async_copy(fpre_hbm.at[sidx.at[i0 + 3]], rows2v, gsem2)
            return _

        lax.fori_loop(0, IBLK // 2 - 1, mbody, None)
        ilast = IBLK - 2
        pltpu.make_async_copy(fpre_hbm.at[sidx.at[ilast]], rowsv,
                              gsem).wait()
        pltpu.sync_copy(rowsv, acc_sh.at[didx.at[ilast]], add=True)
        pltpu.make_async_copy(fpre_hbm.at[sidx.at[ilast + 1]], rows2v,
                              gsem2).wait()
        pltpu.sync_copy(rows2v, acc_sh.at[didx.at[ilast + 1]], add=True)
    plsc.subcore_barrier()

    roff = s * ROWS_PER_TILE

    @pl.when(c == 0)
    def _():
        pltpu.sync_copy(acc_sh.at[pl.ds(roff, ROWS_PER_TILE)],
                        h0_hbm.at[pl.ds(roff, ROWS_PER_TILE)])

    @pl.when(c == 1)
    def _():
        pltpu.sync_copy(acc_sh.at[pl.ds(roff, ROWS_PER_TILE)],
                        h1_hbm.at[pl.ds(roff, ROWS_PER_TILE)])


_ROWBLK = 2048
_GRID = NPAD // _ROWBLK


def _norm_col(norm2d):
    nq = _ROWBLK // D
    e1 = (lax.broadcasted_iota(jnp.int32, (_ROWBLK, nq), 1)
          == lax.broadcasted_iota(jnp.int32, (_ROWBLK, nq), 0) // D
          ).astype(jnp.float32)
    t = jnp.dot(e1, norm2d, preferred_element_type=jnp.float32)
    sel = (lax.broadcasted_iota(jnp.int32, (_ROWBLK, D), 1)
           == lax.broadcasted_iota(jnp.int32, (_ROWBLK, D), 0) % D
           ).astype(jnp.float32)
    return jnp.sum(t * sel, axis=1, keepdims=True)


def _k2_body(hist_ref, feat_ref, fpre_ref, norm_ref):
    deg = jnp.sum(hist_ref[...], axis=0)
    norm2d = lax.rsqrt(jnp.maximum(deg, 1.0))
    norm_ref[...] = norm2d
    fpre_ref[...] = feat_ref[...] * _norm_col(norm2d)


def _k2(hist3, features):
    return pl.pallas_call(
        _k2_body,
        grid=(_GRID,),
        in_specs=[
            pl.BlockSpec((NW, _ROWBLK // D, D), lambda i: (0, i, 0)),
            pl.BlockSpec((_ROWBLK, D), lambda i: (i, 0)),
        ],
        out_specs=[
            pl.BlockSpec((_ROWBLK, D), lambda i: (i, 0)),
            pl.BlockSpec((_ROWBLK // D, D), lambda i: (i, 0)),
        ],
        out_shape=[
            jax.ShapeDtypeStruct((NPAD, D), jnp.float32),
            jax.ShapeDtypeStruct((NPAD // D, D), jnp.float32),
        ],
    )(hist3, features)


def _k4_body(h0_ref, h1_ref, norm_ref, feat_ref, init_ref, aw_ref, out_ref):
    h = (h0_ref[...] + h1_ref[...]) * _norm_col(norm_ref[...])
    f = feat_ref[...]
    a1 = aw_ref[0:1, 0:D]
    a2 = aw_ref[0:1, D:2 * D]
    logit = (jnp.sum(f * a1, axis=1, keepdims=True)
             + jnp.sum(h * a2, axis=1, keepdims=True))
    alpha = jax.nn.sigmoid(logit)
    out_ref[...] = alpha * h + init_ref[...]


def _k4(h0, h1, norm2d, features, initial_features, a_weight):
    blk = lambda i: (i, 0)
    return pl.pallas_call(
        _k4_body,
        grid=(_GRID,),
        in_specs=[
            pl.BlockSpec((_ROWBLK, D), blk),
            pl.BlockSpec((_ROWBLK, D), blk),
            pl.BlockSpec((_ROWBLK // D, D), blk),
            pl.BlockSpec((_ROWBLK, D), blk),
            pl.BlockSpec((_ROWBLK, D), blk),
            pl.BlockSpec((1, 2 * D), lambda i: (0, 0)),
        ],
        out_specs=pl.BlockSpec((_ROWBLK, D), blk),
        out_shape=jax.ShapeDtypeStruct((N, D), jnp.float32),
    )(h0, h1, norm2d, features, initial_features, a_weight)


_sc_mesh = plsc.VectorSubcoreMesh(core_axis_name="c", subcore_axis_name="s")

_k1 = functools.partial(
    pl.kernel,
    out_type=jax.ShapeDtypeStruct((NW, NPAD), jnp.float32),
    mesh=_sc_mesh,
    compiler_params=pltpu.CompilerParams(needs_layout_passes=False),
    scratch_types=[
        pltpu.VMEM((K3_NCHUNK, K3_CHUNK), jnp.int32),
        pltpu.VMEM((NPAD,), jnp.float32),
        pltpu.VMEM((16,), jnp.int32),
    ],
)(_k1_body)

_k3 = functools.partial(
    pl.kernel,
    out_type=(
        jax.ShapeDtypeStruct((NPAD, D), jnp.float32),
        jax.ShapeDtypeStruct((NPAD, D), jnp.float32),
    ),
    mesh=_sc_mesh,
    scratch_types=[
        pltpu.VMEM((IBLK, K3_CHUNK), jnp.int32),
        pltpu.VMEM((IBLK, K3_CHUNK), jnp.int32),
        pltpu.VMEM((K3_CHUNK, D), jnp.float32),
        pltpu.VMEM((K3_CHUNK, D), jnp.float32),
        pltpu.VMEM((16, D), jnp.float32),
        pltpu.VMEM_SHARED((NPAD, D), jnp.float32),
        pltpu.SemaphoreType.DMA,
        pltpu.SemaphoreType.DMA,
    ],
)(_k3_body)


def kernel(features, initial_features, edge_index, a_weight):
    src = edge_index[0]
    dst = edge_index[1]

    pad3 = NW * E3_PER_TILE - E
    dummy_dst = N + jnp.arange(pad3, dtype=jnp.int32) % (NPAD - N)
    dummy_src = jnp.arange(pad3, dtype=jnp.int32) % N
    src3 = jnp.concatenate(
        [src, dummy_src]).reshape(NW, K3_NCHUNK, K3_CHUNK)
    dst3 = jnp.concatenate(
        [dst, dummy_dst]).reshape(NW, K3_NCHUNK, K3_CHUNK)

    hist = _k1(dst3)
    fpre, norm2d = _k2(hist.reshape(NW, NPAD // D, D), features)
    h0, h1 = _k3(src3, dst3, fpre)
    out = _k4(h0, h1, norm2d, features, initial_features, a_weight)
    return out


# speedup vs baseline: 1.1776x; 1.0622x over previous
import functools

import jax
import jax.numpy as jnp
from jax import lax
from jax.experimental import pallas as pl
from jax.experimental.pallas import tpu as pltpu
from jax.experimental.pallas import tpu_sc as plsc

N = 10000
E = 320000
D = 128

NC = 2
NS = 16
NW = NC * NS

NPAD = 10240

K3_CHUNK = 128
K3_NCHUNK = 80
E3_PER_TILE = K3_CHUNK * K3_NCHUNK
IBLK = 16

ROWS_PER_TILE = NPAD // NS


def _zero_1d(ref, nwords):
    zeros = jnp.zeros((16,), ref.dtype)

    def body(i, _):
        ref[pl.ds(i * 16, 16)] = zeros
        return _

    lax.fori_loop(0, nwords // 16, body, None)


def _k1_body(dst_hbm, hist_hbm, dstv, histv):
    c = lax.axis_index("c")
    s = lax.axis_index("s")
    w = s * NC + c

    pltpu.sync_copy(dst_hbm.at[w], dstv)
    _zero_1d(histv, NPAD)
    ones_f = jnp.ones((16,), jnp.float32)

    def chunk_body(j, _):
        for k in range(K3_CHUNK // 16):
            d = dstv[j, pl.ds(k * 16, 16)]
            plsc.addupdate_scatter(histv, [d], ones_f)
        return _

    lax.fori_loop(0, K3_NCHUNK, chunk_body, None)
    pltpu.sync_copy(histv, hist_hbm.at[w])


def _k3_body(src_hbm, dst_hbm, fpre_hbm, h0_hbm, h1_hbm,
             sidx, didx, rowsv, rows2v, zrow, acc_sh, gsem, gsem2):
    c = lax.axis_index("c")
    s = lax.axis_index("s")
    w = s * NC + c

    zeros = jnp.zeros((16,), jnp.float32)
    for i in range(16):
        for k in range(D // 16):
            zrow[i, pl.ds(k * 16, 16)] = zeros

    def zb(b, _):
        pltpu.sync_copy(zrow, acc_sh.at[pl.ds(s * ROWS_PER_TILE + b * 16, 16)])
        return _

    lax.fori_loop(0, ROWS_PER_TILE // 16, zb, None)
    plsc.subcore_barrier()

    for b in range(K3_NCHUNK // IBLK):
        pltpu.sync_copy(src_hbm.at[w, pl.ds(b * IBLK, IBLK)], sidx)
        pltpu.sync_copy(dst_hbm.at[w, pl.ds(b * IBLK, IBLK)], didx)
        pltpu.async_copy(fpre_hbm.at[sidx.at[0]], rowsv, gsem)
        pltpu.async_copy(fpre_hbm.at[sidx.at[1]], rows2v, gsem2)

        def mbody(ii, _):
            i0 = 2 * ii
            pltpu.make_async_copy(fpre_hbm.at[sidx.at[i0]], rowsv,
                                  gsem).wait()
            pltpu.sync_copy(rowsv, acc_sh.at[didx.at[i0]], add=True)
            pltpu.async_copy(fpre_hbm.at[sidx.at[i0 + 2]], rowsv, gsem)
            pltpu.make_async_copy(fpre_hbm.at[sidx.at[i0 + 1]], rows2v,
                                  gsem2).wait()
            pltpu.sync_copy(rows2v, acc_sh.at[didx.at[i0 + 1]], add=True)
            pltpu.async_copy(fpre_hbm.at[sidx.at[i0 + 3]], rows2v, gsem2)
            return _

        lax.fori_loop(0, IBLK // 2 - 1, mbody, None)
        ilast = IBLK - 2
        pltpu.make_async_copy(fpre_hbm.at[sidx.at[ilast]], rowsv,
                              gsem).wait()
        pltpu.sync_copy(rowsv, acc_sh.at[didx.at[ilast]], add=True)
        pltpu.make_async_copy(fpre_hbm.at[sidx.at[ilast + 1]], rows2v,
                              gsem2).wait()
        pltpu.sync_copy(rows2v, acc_sh.at[didx.at[ilast + 1]], add=True)
    plsc.subcore_barrier()

    roff = s * ROWS_PER_TILE

    @pl.when(c == 0)
    def _():
        pltpu.sync_copy(acc_sh.at[pl.ds(roff, ROWS_PER_TILE)],
                        h0_hbm.at[pl.ds(roff, ROWS_PER_TILE)])

    @pl.when(c == 1)
    def _():
        pltpu.sync_copy(acc_sh.at[pl.ds(roff, ROWS_PER_TILE)],
                        h1_hbm.at[pl.ds(roff, ROWS_PER_TILE)])


_ROWBLK = 2048
_GRID = NPAD // _ROWBLK


def _norm_col(norm2d):
    nq = _ROWBLK // D
    e1 = (lax.broadcasted_iota(jnp.int32, (_ROWBLK, nq), 1)
          == lax.broadcasted_iota(jnp.int32, (_ROWBLK, nq), 0) // D
          ).astype(jnp.float32)
    t = jnp.dot(e1, norm2d, preferred_element_type=jnp.float32)
    sel = (lax.broadcasted_iota(jnp.int32, (_ROWBLK, D), 1)
           == lax.broadcasted_iota(jnp.int32, (_ROWBLK, D), 0) % D
           ).astype(jnp.float32)
    return jnp.sum(t * sel, axis=1, keepdims=True)


def _k2_body(hist_ref, feat_ref, fpre_ref, norm_ref):
    deg = jnp.sum(hist_ref[...], axis=0)
    norm2d = lax.rsqrt(jnp.maximum(deg, 1.0))
    norm_ref[...] = norm2d
    fpre_ref[...] = feat_ref[...] * _norm_col(norm2d)


def _k2(hist3, features):
    return pl.pallas_call(
        _k2_body,
        grid=(_GRID,),
        in_specs=[
            pl.BlockSpec((NW, _ROWBLK // D, D), lambda i: (0, i, 0)),
            pl.BlockSpec((_ROWBLK, D), lambda i: (i, 0)),
        ],
        out_specs=[
            pl.BlockSpec((_ROWBLK, D), lambda i: (i, 0)),
            pl.BlockSpec((_ROWBLK // D, D), lambda i: (i, 0)),
        ],
        out_shape=[
            jax.ShapeDtypeStruct((NPAD, D), jnp.float32),
            jax.ShapeDtypeStruct((NPAD // D, D), jnp.float32),
        ],
    )(hist3, features)


def _k4_body(h0_ref, h1_ref, norm_ref, feat_ref, init_ref, aw_ref, out_ref):
    h = (h0_ref[...] + h1_ref[...]) * _norm_col(norm_ref[...])
    f = feat_ref[...]
    a1 = aw_ref[0:1, 0:D]
    a2 = aw_ref[0:1, D:2 * D]
    logit = (jnp.sum(f * a1, axis=1, keepdims=True)
             + jnp.sum(h * a2, axis=1, keepdims=True))
    alpha = jax.nn.sigmoid(logit)
    out_ref[...] = alpha * h + init_ref[...]


def _k4(h0, h1, norm2d, features, initial_features, a_weight):
    blk = lambda i: (i, 0)
    return pl.pallas_call(
        _k4_body,
        grid=(_GRID,),
        in_specs=[
            pl.BlockSpec((_ROWBLK, D), blk),
            pl.BlockSpec((_ROWBLK, D), blk),
            pl.BlockSpec((_ROWBLK // D, D), blk),
            pl.BlockSpec((_ROWBLK, D), blk),
            pl.BlockSpec((_ROWBLK, D), blk),
            pl.BlockSpec((1, 2 * D), lambda i: (0, 0)),
        ],
        out_specs=pl.BlockSpec((_ROWBLK, D), blk),
        out_shape=jax.ShapeDtypeStruct((N, D), jnp.float32),
    )(h0, h1, norm2d, features, initial_features, a_weight)


_sc_mesh = plsc.VectorSubcoreMesh(core_axis_name="c", subcore_axis_name="s")

_k1 = functools.partial(
    pl.kernel,
    out_type=jax.ShapeDtypeStruct((NW, NPAD), jnp.float32),
    mesh=_sc_mesh,
    compiler_params=pltpu.CompilerParams(needs_layout_passes=False),
    scratch_types=[
        pltpu.VMEM((K3_NCHUNK, K3_CHUNK), jnp.int32),
        pltpu.VMEM((NPAD,), jnp.float32),
    ],
)(_k1_body)

_k3 = functools.partial(
    pl.kernel,
    out_type=(
        jax.ShapeDtypeStruct((NPAD, D), jnp.float32),
        jax.ShapeDtypeStruct((NPAD, D), jnp.float32),
    ),
    mesh=_sc_mesh,
    scratch_types=[
        pltpu.VMEM((IBLK, K3_CHUNK), jnp.int32),
        pltpu.VMEM((IBLK, K3_CHUNK), jnp.int32),
        pltpu.VMEM((K3_CHUNK, D), jnp.float32),
        pltpu.VMEM((K3_CHUNK, D), jnp.float32),
        pltpu.VMEM((16, D), jnp.float32),
        pltpu.VMEM_SHARED((NPAD, D), jnp.float32),
        pltpu.SemaphoreType.DMA,
        pltpu.SemaphoreType.DMA,
    ],
)(_k3_body)


def kernel(features, initial_features, edge_index, a_weight):
    src = edge_index[0]
    dst = edge_index[1]

    pad3 = NW * E3_PER_TILE - E
    dummy_dst = N + jnp.arange(pad3, dtype=jnp.int32) % (NPAD - N)
    dummy_src = jnp.arange(pad3, dtype=jnp.int32) % N
    src3 = jnp.concatenate(
        [src, dummy_src]).reshape(NW, K3_NCHUNK, K3_CHUNK)
    dst3 = jnp.concatenate(
        [dst, dummy_dst]).reshape(NW, K3_NCHUNK, K3_CHUNK)

    hist = _k1(dst3)
    fpre, norm2d = _k2(hist.reshape(NW, NPAD // D, D), features)
    h0, h1 = _k3(src3, dst3, fpre)
    out = _k4(h0, h1, norm2d, features, initial_features, a_weight)
    return out


# speedup vs baseline: 1.1904x; 1.0108x over previous
import functools

import jax
import jax.numpy as jnp
from jax import lax
from jax.experimental import pallas as pl
from jax.experimental.pallas import tpu as pltpu
from jax.experimental.pallas import tpu_sc as plsc

N = 10000
E = 320000
D = 128

NC = 2
NS = 16
NW = NC * NS

NPAD = 10240

K3_CHUNK = 128
K3_NCHUNK = 80
E3_PER_TILE = K3_CHUNK * K3_NCHUNK
IBLK = 16

ROWS_PER_TILE = NPAD // NS


def _zero_1d(ref, nwords):
    zeros = jnp.zeros((16,), ref.dtype)

    def body(i, _):
        ref[pl.ds(i * 16, 16)] = zeros
        return _

    lax.fori_loop(0, nwords // 16, body, None)


def _k1_body(dst_hbm, hist_hbm, dstv, histv):
    c = lax.axis_index("c")
    s = lax.axis_index("s")
    w = s * NC + c

    pltpu.sync_copy(dst_hbm.at[w], dstv)
    _zero_1d(histv, NPAD)
    ones_f = jnp.ones((16,), jnp.float32)

    def chunk_body(j, _):
        for k in range(K3_CHUNK // 16):
            d = dstv[j, pl.ds(k * 16, 16)]
            plsc.addupdate_scatter(histv, [d], ones_f)
        return _

    lax.fori_loop(0, K3_NCHUNK, chunk_body, None)
    pltpu.sync_copy(histv, hist_hbm.at[w])


def _k3_body(src_hbm, dst_hbm, fpre_hbm, h0_hbm, h1_hbm,
             sidx, didx, rowsv, rows2v, zrow, acc_sh, gsem, gsem2):
    c = lax.axis_index("c")
    s = lax.axis_index("s")
    w = s * NC + c

    zeros = jnp.zeros((16,), jnp.float32)
    for i in range(16):
        for k in range(D // 16):
            zrow[i, pl.ds(k * 16, 16)] = zeros
    NZB = 10
    for g in range(ROWS_PER_TILE // 16 // NZB):
        for b in range(NZB):
            roff = s * ROWS_PER_TILE + (g * NZB + b) * 16
            pltpu.async_copy(zrow, acc_sh.at[pl.ds(roff, 16)], gsem)
        for b in range(NZB):
            roff = s * ROWS_PER_TILE + (g * NZB + b) * 16
            pltpu.make_async_copy(zrow, acc_sh.at[pl.ds(roff, 16)],
                                  gsem).wait()
    plsc.subcore_barrier()

    for b in range(K3_NCHUNK // IBLK):
        pltpu.sync_copy(src_hbm.at[w, pl.ds(b * IBLK, IBLK)], sidx)
        pltpu.sync_copy(dst_hbm.at[w, pl.ds(b * IBLK, IBLK)], didx)
        pltpu.async_copy(fpre_hbm.at[sidx.at[0]], rowsv, gsem)
        pltpu.async_copy(fpre_hbm.at[sidx.at[1]], rows2v, gsem2)

        def mbody(ii, _):
            i0 = 2 * ii
            pltpu.make_async_copy(fpre_hbm.at[sidx.at[i0]], rowsv,
                                  gsem).wait()
            pltpu.sync_copy(rowsv, acc_sh.at[didx.at[i0]], add=True)
            pltpu.async_copy(fpre_hbm.at[sidx.at[i0 + 2]], rowsv, gsem)
            pltpu.make_async_copy(fpre_hbm.at[sidx.at[i0 + 1]], rows2v,
                                  gsem2).wait()
            pltpu.sync_copy(rows2v, acc_sh.at[didx.at[i0 + 1]], add=True)
            pltpu.async_copy(fpre_hbm.at[sidx.at[i0 + 3]], rows2v, gsem2)
            return _

        lax.fori_loop(0, IBLK // 2 - 1, mbody, None)
        ilast = IBLK - 2
        pltpu.make_async_copy(fpre_hbm.at[sidx.at[ilast]], rowsv,
                              gsem).wait()
        pltpu.sync_copy(rowsv, acc_sh.at[didx.at[ilast]], add=True)
        pltpu.make_async_copy(fpre_hbm.at[sidx.at[ilast + 1]], rows2v,
                              gsem2).wait()
        pltpu.sync_copy(rows2v, acc_sh.at[didx.at[ilast + 1]], add=True)
    plsc.subcore_barrier()

    roff = s * ROWS_PER_TILE

    @pl.when(c == 0)
    def _():
        pltpu.sync_copy(acc_sh.at[pl.ds(roff, ROWS_PER_TILE)],
                        h0_hbm.at[pl.ds(roff, ROWS_PER_TILE)])

    @pl.when(c == 1)
    def _():
        pltpu.sync_copy(acc_sh.at[pl.ds(roff, ROWS_PER_TILE)],
                        h1_hbm.at[pl.ds(roff, ROWS_PER_TILE)])


_ROWBLK = 2048
_GRID = NPAD // _ROWBLK


def _norm_col(norm2d):
    nq = _ROWBLK // D
    e1 = (lax.broadcasted_iota(jnp.int32, (_ROWBLK, nq), 1)
          == lax.broadcasted_iota(jnp.int32, (_ROWBLK, nq), 0) // D
          ).astype(jnp.float32)
    t = jnp.dot(e1, norm2d, preferred_element_type=jnp.float32)
    sel = (lax.broadcasted_iota(jnp.int32, (_ROWBLK, D), 1)
           == lax.broadcasted_iota(jnp.int32, (_ROWBLK, D), 0) % D
           ).astype(jnp.float32)
    return jnp.sum(t * sel, axis=1, keepdims=True)


def _k2_body(hist_ref, feat_ref, fpre_ref, norm_ref):
    deg = jnp.sum(hist_ref[...], axis=0)
    norm2d = lax.rsqrt(jnp.maximum(deg, 1.0))
    norm_ref[...] = norm2d
    fpre_ref[...] = feat_ref[...] * _norm_col(norm2d)


def _k2(hist3, features):
    return pl.pallas_call(
        _k2_body,
        grid=(_GRID,),
        in_specs=[
            pl.BlockSpec((NW, _ROWBLK // D, D), lambda i: (0, i, 0)),
            pl.BlockSpec((_ROWBLK, D), lambda i: (i, 0)),
        ],
        out_specs=[
            pl.BlockSpec((_ROWBLK, D), lambda i: (i, 0)),
            pl.BlockSpec((_ROWBLK // D, D), lambda i: (i, 0)),
        ],
        out_shape=[
            jax.ShapeDtypeStruct((NPAD, D), jnp.float32),
            jax.ShapeDtypeStruct((NPAD // D, D), jnp.float32),
        ],
    )(hist3, features)


def _k4_body(h0_ref, h1_ref, norm_ref, feat_ref, init_ref, aw_ref, out_ref):
    h = (h0_ref[...] + h1_ref[...]) * _norm_col(norm_ref[...])
    f = feat_ref[...]
    a1 = aw_ref[0:1, 0:D]
    a2 = aw_ref[0:1, D:2 * D]
    logit = (jnp.sum(f * a1, axis=1, keepdims=True)
             + jnp.sum(h * a2, axis=1, keepdims=True))
    alpha = jax.nn.sigmoid(logit)
    out_ref[...] = alpha * h + init_ref[...]


def _k4(h0, h1, norm2d, features, initial_features, a_weight):
    blk = lambda i: (i, 0)
    return pl.pallas_call(
        _k4_body,
        grid=(_GRID,),
        in_specs=[
            pl.BlockSpec((_ROWBLK, D), blk),
            pl.BlockSpec((_ROWBLK, D), blk),
            pl.BlockSpec((_ROWBLK // D, D), blk),
            pl.BlockSpec((_ROWBLK, D), blk),
            pl.BlockSpec((_ROWBLK, D), blk),
            pl.BlockSpec((1, 2 * D), lambda i: (0, 0)),
        ],
        out_specs=pl.BlockSpec((_ROWBLK, D), blk),
        out_shape=jax.ShapeDtypeStruct((N, D), jnp.float32),
    )(h0, h1, norm2d, features, initial_features, a_weight)


_sc_mesh = plsc.VectorSubcoreMesh(core_axis_name="c", subcore_axis_name="s")

_k1 = functools.partial(
    pl.kernel,
    out_type=jax.ShapeDtypeStruct((NW, NPAD), jnp.float32),
    mesh=_sc_mesh,
    compiler_params=pltpu.CompilerParams(needs_layout_passes=False),
    scratch_types=[
        pltpu.VMEM((K3_NCHUNK, K3_CHUNK), jnp.int32),
        pltpu.VMEM((NPAD,), jnp.float32),
    ],
)(_k1_body)

_k3 = functools.partial(
    pl.kernel,
    out_type=(
        jax.ShapeDtypeStruct((NPAD, D), jnp.float32),
        jax.ShapeDtypeStruct((NPAD, D), jnp.float32),
    ),
    mesh=_sc_mesh,
    scratch_types=[
        pltpu.VMEM((IBLK, K3_CHUNK), jnp.int32),
        pltpu.VMEM((IBLK, K3_CHUNK), jnp.int32),
        pltpu.VMEM((K3_CHUNK, D), jnp.float32),
        pltpu.VMEM((K3_CHUNK, D), jnp.float32),
        pltpu.VMEM((16, D), jnp.float32),
        pltpu.VMEM_SHARED((NPAD, D), jnp.float32),
        pltpu.SemaphoreType.DMA,
        pltpu.SemaphoreType.DMA,
    ],
)(_k3_body)


def kernel(features, initial_features, edge_index, a_weight):
    src = edge_index[0]
    dst = edge_index[1]

    pad3 = NW * E3_PER_TILE - E
    dummy_dst = N + jnp.arange(pad3, dtype=jnp.int32) % (NPAD - N)
    dummy_src = jnp.arange(pad3, dtype=jnp.int32) % N
    src3 = jnp.concatenate(
        [src, dummy_src]).reshape(NW, K3_NCHUNK, K3_CHUNK)
    dst3 = jnp.concatenate(
        [dst, dummy_dst]).reshape(NW, K3_NCHUNK, K3_CHUNK)

    hist = _k1(dst3)
    fpre, norm2d = _k2(hist.reshape(NW, NPAD // D, D), features)
    h0, h1 = _k3(src3, dst3, fpre)
    out = _k4(h0, h1, norm2d, features, initial_features, a_weight)
    return out
